# Initial kernel scaffold; baseline (speedup 1.0000x reference)
import jax
import jax.numpy as jnp
from jax.experimental import pallas as pl


def kernel(x, v, C, F, Jp, material, gravity, attractor_strength, attractor_pos):
    raise NotImplementedError("write your pallas kernel here")


# speedup vs baseline: 475.2303x; 475.2303x over previous
import functools
import jax
import jax.numpy as jnp
from jax import lax
from jax.experimental import pallas as pl
from jax.experimental.pallas import tpu as pltpu
from jax.experimental.pallas import tpu_sc as plsc

_N_P = 18000
_N_GRID = 128
_DX = 1.0 / _N_GRID
_INV_DX = float(_N_GRID)
_DT = 1e-4
_P_VOL = (_DX * 0.5) ** 2
_P_MASS = _P_VOL * 1.0
_E_MOD = 5000.0
_NU = 0.2
_MU_0 = _E_MOD / (2 * (1 + _NU))
_LA_0 = _E_MOD * _NU / ((1 + _NU) * (1 - 2 * _NU))

_NW = 32
_PPW = 576
_NPP = _NW * _PPW
_NCHUNK = _PPW // 16
_NNODE = _N_GRID * _N_GRID
_NPT = _NNODE // 16

_mesh = plsc.VectorSubcoreMesh(core_axis_name="c", subcore_axis_name="s")
_cparams = pltpu.CompilerParams(needs_layout_passes=False)
_f32 = jnp.float32
_i32 = jnp.int32


def _rsqrt(x):
    xi = lax.bitcast_convert_type(x, _i32)
    yi = jnp.int32(0x5F3759DF) - (xi >> 1)
    y = lax.bitcast_convert_type(yi, _f32)
    for _ in range(3):
        y = y * (1.5 - 0.5 * x * y * y)
    return y


def _inv(x):
    r = _rsqrt(x)
    return r * r


def _bf16r(x):
    xi = lax.bitcast_convert_type(x, _i32)
    r = (xi + 0x7FFF + ((xi >> 16) & 1)) & jnp.int32(-65536)
    return lax.bitcast_convert_type(r, _f32)


def _spline_w(f):
    t0 = 1.5 - f
    t1 = f - 1.0
    t2 = f - 0.5
    return [0.5 * t0 * t0, 0.75 - t1 * t1, 0.5 * t2 * t2]


@functools.partial(
    pl.kernel,
    out_type=(
        jax.ShapeDtypeStruct((_NW, 3 * _NNODE), _f32),
        jax.ShapeDtypeStruct((_NPP,), _f32),
        jax.ShapeDtypeStruct((_NPP,), _f32),
        jax.ShapeDtypeStruct((_NPP,), _f32),
        jax.ShapeDtypeStruct((_NPP,), _f32),
        jax.ShapeDtypeStruct((_NPP,), _f32),
    ),
    mesh=_mesh,
    scratch_types=(
        [pltpu.VMEM((_PPW,), _f32)] * 13
        + [pltpu.VMEM((_PPW,), _i32)]
        + [pltpu.VMEM((_NNODE,), _f32)] * 3
        + [pltpu.VMEM((_PPW,), _f32)] * 5
    ),
    compiler_params=_cparams,
)
def _p2g(x0, x1, v0, v1, c00, c01, c10, c11, f00, f01, f10, f11, jp, mt,
         pg, of00, of01, of10, of11, ojp,
         sx0, sx1, sv0, sv1, sc00, sc01, sc10, sc11, sf00, sf01, sf10, sf11,
         sjp, smt, g0, g1, g2, t00, t01, t10, t11, tjp):
    cid = lax.axis_index("c")
    sid = lax.axis_index("s")
    wid = cid * 16 + sid
    base = wid * _PPW

    for src, dst in ((x0, sx0), (x1, sx1), (v0, sv0), (v1, sv1),
                     (c00, sc00), (c01, sc01), (c10, sc10), (c11, sc11),
                     (f00, sf00), (f01, sf01), (f10, sf10), (f11, sf11),
                     (jp, sjp), (mt, smt)):
        pltpu.sync_copy(src.at[pl.ds(base, _PPW)], dst)

    zer = jnp.zeros((16,), _f32)

    def zbody(i, carry):
        sl = pl.ds(i * 16, 16)
        g0[sl] = zer
        g1[sl] = zer
        g2[sl] = zer
        return carry

    lax.fori_loop(0, _NPT, zbody, 0)

    lane = lax.iota(_i32, 16)

    def chunk(k, carry):
        sl = pl.ds(k * 16, 16)
        xx = sx0[sl]
        xy = sx1[sl]
        xv0 = sv0[sl]
        xv1 = sv1[sl]
        xc00 = sc00[sl]
        xc01 = sc01[sl]
        xc10 = sc10[sl]
        xc11 = sc11[sl]
        xjp = sjp[sl]
        xmt = smt[sl]
        valid = (base + k * 16 + lane) < _N_P

        bf00 = _bf16r(sf00[sl])
        bf01 = _bf16r(sf01[sl])
        bf10 = _bf16r(sf10[sl])
        bf11 = _bf16r(sf11[sl])
        g00 = _bf16r(1.0 + _DT * xc00)
        g01 = _bf16r(_DT * xc01)
        g10 = _bf16r(_DT * xc10)
        g11 = _bf16r(1.0 + _DT * xc11)
        a = g00 * bf00 + g01 * bf10
        b = g00 * bf01 + g01 * bf11
        c = g10 * bf00 + g11 * bf10
        d = g10 * bf01 + g11 * bf11

        is0 = xmt == 0
        is1 = xmt == 1
        is2 = xmt == 2
        h = jnp.clip(jnp.exp(10.0 * (1.0 - xjp)), 0.1, 5.0)
        h = jnp.where(is1, 0.3, h)
        mu = jnp.where(is0, 0.0, _MU_0 * h)
        la = _LA_0 * h

        h2 = (a + d) * (a + d) + (c - b) * (c - b)
        inv_hn = _rsqrt(jnp.maximum(h2, 1e-30))
        ct = (a + d) * inv_hn
        st = (c - b) * inv_hn
        Sq = ct * b + st * d
        mm = 0.5 * (ct * (a + d) + st * (c - b))
        dd = 0.5 * (ct * (a - d) + st * (c + b))
        d2 = dd * dd + Sq * Sq
        delta = d2 * _rsqrt(jnp.maximum(d2, 1e-30))
        s1 = mm + delta
        s2 = mm - delta
        c1x = Sq
        c1y = delta - dd
        c2x = delta + dd
        c2y = Sq
        n1 = c1x * c1x + c1y * c1y
        n2 = c2x * c2x + c2y * c2y
        use1 = n1 >= n2
        evx = jnp.where(use1, c1x, c2x)
        evy = jnp.where(use1, c1y, c2y)
        nn = evx * evx + evy * evy
        degen = nn < 1e-30
        evx = jnp.where(degen, 1.0, evx)
        evy = jnp.where(degen, 0.0, evy)
        nn = jnp.where(degen, 1.0, nn)
        innr = _rsqrt(nn)
        evx = evx * innr
        evy = evy * innr

        ns1 = jnp.where(is2, jnp.clip(s1, 1.0 - 2.5e-2, 1.0 + 4.5e-3), s1)
        ns2 = jnp.where(is2, jnp.clip(s2, 1.0 - 2.5e-2, 1.0 + 4.5e-3), s2)
        jp_new = xjp * (s1 * _inv(jnp.maximum(ns1, 1e-30))) \
                     * (s2 * _inv(jnp.maximum(ns2, 1e-30)))
        J = ns1 * ns2
        cu = ct * evx - st * evy
        su = st * evx + ct * evy
        bcu = _bf16r(cu)
        bsu = _bf16r(su)
        bvx = _bf16r(evx)
        bvy = _bf16r(evy)
        R00 = bcu * bvx + bsu * bvy
        R01 = bcu * bvy - bsu * bvx
        R10 = bsu * bvx - bcu * bvy
        R11 = bsu * bvy + bcu * bvx
        bm00 = _bf16r(ns1 * evx)
        bm01 = _bf16r(ns1 * evy)
        bm10 = _bf16r(-(ns2 * evy))
        bm11 = _bf16r(ns2 * evx)
        Fs_a = bcu * bm00 - bsu * bm10
        Fs_b = bcu * bm01 - bsu * bm11
        Fs_c = bsu * bm00 + bcu * bm10
        Fs_d = bsu * bm01 + bcu * bm11
        sqJ = J * _rsqrt(jnp.maximum(J, 1e-30))
        Fa = jnp.where(is0, sqJ, jnp.where(is2, Fs_a, a))
        Fb = jnp.where(is0, 0.0, jnp.where(is2, Fs_b, b))
        Fc = jnp.where(is0, 0.0, jnp.where(is2, Fs_c, c))
        Fd = jnp.where(is0, sqJ, jnp.where(is2, Fs_d, d))

        k2mu = 2.0 * mu
        laJ = la * J * (J - 1.0)
        e00 = _bf16r(Fa - R00)
        e01 = _bf16r(Fb - R01)
        e10 = _bf16r(Fc - R10)
        e11 = _bf16r(Fd - R11)
        tb00 = _bf16r(Fa)
        tb01 = _bf16r(Fb)
        tb10 = _bf16r(Fc)
        tb11 = _bf16r(Fd)
        scl = -_DT * _P_VOL * 4.0 * _INV_DX * _INV_DX
        A11 = scl * (k2mu * (e00 * tb00 + e01 * tb01) + laJ) + _P_MASS * xc00
        A12 = scl * (k2mu * (e00 * tb10 + e01 * tb11)) + _P_MASS * xc01
        A21 = scl * (k2mu * (e10 * tb00 + e11 * tb01)) + _P_MASS * xc10
        A22 = scl * (k2mu * (e10 * tb10 + e11 * tb11) + laJ) + _P_MASS * xc11
        ba11 = _bf16r(A11)
        ba12 = _bf16r(A12)
        ba21 = _bf16r(A21)
        ba22 = _bf16r(A22)

        t00[sl] = Fa
        t01[sl] = Fb
        t10[sl] = Fc
        t11[sl] = Fd
        tjp[sl] = jp_new

        bx = (xx * _INV_DX - 0.5).astype(_i32)
        by = (xy * _INV_DX - 0.5).astype(_i32)
        fx0 = xx * _INV_DX - bx.astype(_f32)
        fx1 = xy * _INV_DX - by.astype(_f32)
        w0 = _spline_w(fx0)
        w1 = _spline_w(fx1)
        mv0 = _P_MASS * xv0
        mv1 = _P_MASS * xv1
        for i in range(3):
            bdx = _bf16r((i - fx0) * _DX)
            lini = (bx + i) * _N_GRID
            for j in range(3):
                bdy = _bf16r((j - fx1) * _DX)
                weight = w0[i] * w1[j]
                lin = lini + by + j
                cvx = weight * (mv0 + (ba11 * bdx + ba12 * bdy))
                cvy = weight * (mv1 + (ba21 * bdx + ba22 * bdy))
                plsc.addupdate_scatter(g0, [lin], cvx, mask=valid)
                plsc.addupdate_scatter(g1, [lin], cvy, mask=valid)
                plsc.addupdate_scatter(g2, [lin], weight * _P_MASS, mask=valid)
        return carry

    lax.fori_loop(0, _NCHUNK, chunk, 0)

    pltpu.sync_copy(g0, pg.at[wid, pl.ds(0, _NNODE)])
    pltpu.sync_copy(g1, pg.at[wid, pl.ds(_NNODE, _NNODE)])
    pltpu.sync_copy(g2, pg.at[wid, pl.ds(2 * _NNODE, _NNODE)])
    pltpu.sync_copy(t00, of00.at[pl.ds(base, _PPW)])
    pltpu.sync_copy(t01, of01.at[pl.ds(base, _PPW)])
    pltpu.sync_copy(t10, of10.at[pl.ds(base, _PPW)])
    pltpu.sync_copy(t11, of11.at[pl.ds(base, _PPW)])
    pltpu.sync_copy(tjp, ojp.at[pl.ds(base, _PPW)])


@functools.partial(
    pl.kernel,
    out_type=tuple(jax.ShapeDtypeStruct((_NPP,), _f32) for _ in range(8)),
    mesh=_mesh,
    scratch_types=(
        [pltpu.VMEM((_NW, _NPT), _f32)]
        + [pltpu.VMEM((_NPT,), _f32)] * 3
        + [pltpu.VMEM_SHARED((_NNODE,), _f32)] * 2
        + [pltpu.VMEM((_NNODE,), _f32)] * 2
        + [pltpu.VMEM((_PPW,), _f32)] * 2
        + [pltpu.VMEM((8, 16), _f32)]
        + [pltpu.VMEM((_PPW,), _f32)] * 8
    ),
    compiler_params=_cparams,
)
def _g2p(pg, px0, px1, par,
         oxn0, oxn1, onv0, onv1, oc00, oc01, oc10, oc11,
         red, acc0, acc1, acc2, shx, shy, gfx, gfy, sx0, sx1, spar,
         u0, u1, u2, u3, u4, u5, u6, u7):
    cid = lax.axis_index("c")
    sid = lax.axis_index("s")
    wid = cid * 16 + sid
    pbase = wid * _PPW
    nodeoff = sid * _NPT

    pltpu.sync_copy(par, spar)
    pltpu.sync_copy(px0.at[pl.ds(pbase, _PPW)], sx0)
    pltpu.sync_copy(px1.at[pl.ds(pbase, _PPW)], sx1)

    for comp, acc in ((0, acc0), (1, acc1), (2, acc2)):
        pltpu.sync_copy(pg.at[:, pl.ds(comp * _NNODE + nodeoff, _NPT)], red)

        def rbody(i, carry):
            sl = pl.ds(i * 16, 16)
            s = red[0, sl]
            for w in range(1, _NW):
                s = s + red[w, sl]
            acc[sl] = s
            return carry

        lax.fori_loop(0, _NPT // 16, rbody, 0)

    lane = lax.iota(_i32, 16)

    def ubody(i, carry):
        sl = pl.ds(i * 16, 16)
        n = nodeoff + i * 16 + lane
        mrow = acc2[sl]
        havem = mrow > 0.0
        invm = _inv(jnp.maximum(mrow, 1e-30))
        gx = acc0[sl] * invm + spar[3]
        gy = acc1[sl] * invm + spar[4]
        ii = n >> 7
        jj = n & 127
        dx_ = spar[0] - _DX * ii.astype(_f32)
        dy_ = spar[1] - _DX * jj.astype(_f32)
        dn2 = dx_ * dx_ + dy_ * dy_
        dn = dn2 * _rsqrt(jnp.maximum(dn2, 1e-30))
        fac = spar[2] * _inv(dn + 0.01)
        gx = gx + dx_ * fac
        gy = gy + dy_ * fac
        gx = jnp.where((ii < 3) & (gx < 0.0), 0.0, gx)
        gx = jnp.where((ii > _N_GRID - 3) & (gx > 0.0), 0.0, gx)
        gy = jnp.where((jj < 3) & (gy < 0.0), 0.0, gy)
        gy = jnp.where((jj > _N_GRID - 3) & (gy > 0.0), 0.0, gy)
        acc0[sl] = jnp.where(havem, gx, 0.0)
        acc1[sl] = jnp.where(havem, gy, 0.0)
        return carry

    lax.fori_loop(0, _NPT // 16, ubody, 0)

    pltpu.sync_copy(acc0, shx.at[pl.ds(nodeoff, _NPT)])
    pltpu.sync_copy(acc1, shy.at[pl.ds(nodeoff, _NPT)])
    plsc.subcore_barrier()
    pltpu.sync_copy(shx, gfx)
    pltpu.sync_copy(shy, gfy)

    def chunk(k, carry):
        sl = pl.ds(k * 16, 16)
        xx = sx0[sl]
        xy = sx1[sl]
        bx = (xx * _INV_DX - 0.5).astype(_i32)
        by = (xy * _INV_DX - 0.5).astype(_i32)
        fx0 = xx * _INV_DX - bx.astype(_f32)
        fx1 = xy * _INV_DX - by.astype(_f32)
        w0 = _spline_w(fx0)
        w1 = _spline_w(fx1)
        zv = jnp.zeros((16,), _f32)
        nvx = zv
        nvy = zv
        nc00 = zv
        nc01 = zv
        nc10 = zv
        nc11 = zv
        for i in range(3):
            dpx = i - fx0
            lini = (bx + i) * _N_GRID
            for j in range(3):
                dpy = j - fx1
                lin = lini + by + j
                gx = plsc.load_gather(gfx, [lin])
                gy = plsc.load_gather(gfy, [lin])
                weight = w0[i] * w1[j]
                k4 = (4.0 * _INV_DX) * weight
                nvx = nvx + weight * gx
                nvy = nvy + weight * gy
                nc00 = nc00 + k4 * gx * dpx
                nc01 = nc01 + k4 * gx * dpy
                nc10 = nc10 + k4 * gy * dpx
                nc11 = nc11 + k4 * gy * dpy
        u0[sl] = xx + _DT * nvx
        u1[sl] = xy + _DT * nvy
        u2[sl] = nvx
        u3[sl] = nvy
        u4[sl] = nc00
        u5[sl] = nc01
        u6[sl] = nc10
        u7[sl] = nc11
        return carry

    lax.fori_loop(0, _NCHUNK, chunk, 0)

    for src, dst in ((u0, oxn0), (u1, oxn1), (u2, onv0), (u3, onv1),
                     (u4, oc00), (u5, oc01), (u6, oc10), (u7, oc11)):
        pltpu.sync_copy(src, dst.at[pl.ds(pbase, _PPW)])


def kernel(x, v, C, F, Jp, material, gravity, attractor_strength, attractor_pos):
    pad = _NPP - _N_P

    def padf(arr, cv=0.0):
        return jnp.pad(arr.astype(_f32), (0, pad), constant_values=cv)

    x0 = padf(x[:, 0], 0.5)
    x1 = padf(x[:, 1], 0.5)
    v0 = padf(v[:, 0])
    v1 = padf(v[:, 1])
    c00 = padf(C[:, 0, 0])
    c01 = padf(C[:, 0, 1])
    c10 = padf(C[:, 1, 0])
    c11 = padf(C[:, 1, 1])
    f00 = padf(F[:, 0, 0], 1.0)
    f01 = padf(F[:, 0, 1])
    f10 = padf(F[:, 1, 0])
    f11 = padf(F[:, 1, 1], 1.0)
    jp = padf(Jp, 1.0)
    mt = jnp.pad(material.astype(_i32), (0, pad), constant_values=1)

    pg, of00, of01, of10, of11, ojp = _p2g(
        x0, x1, v0, v1, c00, c01, c10, c11, f00, f01, f10, f11, jp, mt)

    gf = gravity.astype(_f32)
    par = jnp.stack([
        jnp.full((16,), attractor_pos[0], _f32),
        jnp.full((16,), attractor_pos[1], _f32),
        jnp.full((16,), attractor_strength * jnp.float32(_DT * 100.0), _f32),
        jnp.full((16,), gf[0] * jnp.float32(_DT * 30.0), _f32),
        jnp.full((16,), gf[1] * jnp.float32(_DT * 30.0), _f32),
        jnp.zeros((16,), _f32),
        jnp.zeros((16,), _f32),
        jnp.zeros((16,), _f32),
    ])

    xn0, xn1, nv0, nv1, nc00, nc01, nc10, nc11 = _g2p(pg, x0, x1, par)

    out = jnp.stack([xn0, xn1, nv0, nv1, nc00, nc01, nc10, nc11,
                     of00, of01, of10, of11, ojp], axis=1)
    return out[:_N_P]
